# baseline (device time: 918421 ns/iter reference)
import numpy as _np

import jax
import jax.numpy as jnp
from jax import lax
from jax.experimental import pallas as pl
from jax.experimental.pallas import tpu as pltpu

N_DEV = 4
SQ = 2048
SKV = 2048
D_MODEL = 1024
H_PER = 8
DH = 128
BLK = 64
NBLK = SKV // BLK
SCALE = 0.08838834764831843

_G = [[b for b in range(NBLK) if b % 3 == r] for r in range(3)]
_P = [_G[0] + [0], _G[1] + [0], _G[2] + [0, 0]]
_PERM_BLOCKS = _P[0] + _P[1] + _P[2]
Q_PERM = _np.concatenate(
    [_np.arange(b * BLK, (b + 1) * BLK) for b in _PERM_BLOCKS]
)
_first: dict[int, int] = {}
for _pos, _rw in enumerate(Q_PERM):
    _first.setdefault(int(_rw), _pos)
INV_PERM = _np.array([_first[i] for i in range(SQ)])

SQP = len(Q_PERM)
CHUNK = SQP // N_DEV
HALF = CHUNK // 2

K_RES_SEL = _np.concatenate(
    [_np.arange(b * BLK, (b + 1) * BLK) for b in _G[0] + _G[1] + _G[2]]
)
SKV_RES = 2048 + 2 * BLK
M_LO = [0, 1408, 704]
M_W = [704, 640, 704]
D_OFF = [0, 704, 1408]
GRP_START = [0, 768, 1536]

SEGS = {
    0: [(0, 576, 0)],
    1: [(576, 192, 0), (768, 384, 1)],
    2: [(1152, 384, 1), (1536, 192, 2)],
    3: [(1728, 576, 2)],
}


def _fused_body(x_ref, wq_ref, k_ref, v_ref, wo_ref,
                out_ref, comm_ref, send_sems, recv_sems, credit_sem):
    my = lax.axis_index("i")
    right = (my + 1) % N_DEV
    left = (my - 1) % N_DEV

    def seg_compute(i0, R, r):
        m_lo, m_w = M_LO[r], M_W[r]
        d_lo = D_OFF[r] + (i0 - GRP_START[r])
        nb = R // BLK
        xb = x_ref[i0:i0 + R, :]
        out_ref[i0:i0 + R, :] = jnp.zeros((R, D_MODEL), jnp.float32)

        def h_body(h, carry):
            q = jnp.dot(xb, wq_ref[h], preferred_element_type=jnp.float32)
            kg = k_ref[h, m_lo:m_lo + m_w, :]
            vg = v_ref[h, m_lo:m_lo + m_w, :]
            sA = lax.dot_general(
                q, kg, (((1,), (1,)), ((), ())),
                preferred_element_type=jnp.float32,
            )
            eA = jnp.exp(sA)
            denom = jnp.sum(eA, axis=1, keepdims=True)
            ctx = jnp.dot(eA, vg, preferred_element_type=jnp.float32)
            if r != 0:
                s0 = lax.dot_general(
                    q, k_ref[h, 0:BLK, :], (((1,), (1,)), ((), ())),
                    preferred_element_type=jnp.float32,
                )
                e0 = jnp.exp(s0)
                denom = denom + jnp.sum(e0, axis=1, keepdims=True)
                ctx = ctx + jnp.dot(e0, v_ref[h, 0:BLK, :],
                                    preferred_element_type=jnp.float32)
                qd = q.reshape(nb, BLK, DH)
                kd = k_ref[h, d_lo:d_lo + R, :].reshape(nb, BLK, DH)
                vd = v_ref[h, d_lo:d_lo + R, :].reshape(nb, BLK, DH)
                sB = lax.dot_general(
                    qd, kd, (((2,), (2,)), ((0,), (0,))),
                    preferred_element_type=jnp.float32,
                )
                eB = jnp.exp(sB)
                denom = denom + jnp.sum(eB, axis=2).reshape(R, 1)
                ctx = ctx + lax.dot_general(
                    eB, vd, (((2,), (1,)), ((0,), (0,))),
                    preferred_element_type=jnp.float32,
                ).reshape(R, DH)
            ctx = ctx / denom
            out_ref[i0:i0 + R, :] += jnp.dot(
                ctx, wo_ref[h], preferred_element_type=jnp.float32
            )
            return carry

        lax.fori_loop(0, H_PER, h_body, None)

    def compute_chunk(c):
        for kk in range(N_DEV):
            @pl.when(c == kk)
            def _(kk=kk):
                for (i0, R, r) in SEGS[kk]:
                    seg_compute(i0, R, r)

    sends = []

    c0 = (my + 3) % N_DEV
    compute_chunk(c0)
    send0 = pltpu.make_async_remote_copy(
        src_ref=out_ref.at[pl.ds(c0 * CHUNK, CHUNK), :],
        dst_ref=comm_ref.at[0],
        send_sem=send_sems.at[0],
        recv_sem=recv_sems.at[0],
        device_id=(right,),
        device_id_type=pl.DeviceIdType.MESH,
    )
    send0.start()
    sends.append(send0)

    for s in range(1, N_DEV):
        c = (my + 3 - s) % N_DEV
        compute_chunk(c)
        rows = pl.ds(c * CHUNK, CHUNK)
        slot = (s - 1) % 2
        recv = pltpu.make_async_remote_copy(
            src_ref=comm_ref.at[slot],
            dst_ref=comm_ref.at[slot],
            send_sem=send_sems.at[slot],
            recv_sem=recv_sems.at[slot],
            device_id=(left,),
            device_id_type=pl.DeviceIdType.MESH,
        )
        recv.wait_recv()
        out_ref[rows, :] += comm_ref[slot]
        if s == 1:
            pl.semaphore_signal(
                credit_sem, inc=1,
                device_id=(left,), device_id_type=pl.DeviceIdType.MESH,
            )
        if s < N_DEV - 1:
            if s == 2:
                pl.semaphore_wait(credit_sem, 1)
                send0.wait_send()
            snd = pltpu.make_async_remote_copy(
                src_ref=out_ref.at[rows, :],
                dst_ref=comm_ref.at[s % 2],
                send_sem=send_sems.at[s % 2],
                recv_sem=recv_sems.at[s % 2],
                device_id=(right,),
                device_id_type=pl.DeviceIdType.MESH,
            )
            snd.start()
            sends.append(snd)

    sends[1].wait_send()
    sends[2].wait_send()


def _ag_body(p_ref, out_ref, sr_send, sr_recv, sl_send, sl_recv):
    my = lax.axis_index("i")
    right = (my + 1) % N_DEV
    left = (my + 3) % N_DEV

    out_ref[...] = p_ref[...]
    for t in range(N_DEV - 1):
        cr = (my - t) % N_DEV
        cl = (my + t) % N_DEV
        slot = t % 2
        ra = pltpu.make_async_remote_copy(
            src_ref=out_ref.at[pl.ds(cr * CHUNK, HALF), :],
            dst_ref=out_ref.at[pl.ds(cr * CHUNK, HALF), :],
            send_sem=sr_send.at[slot],
            recv_sem=sr_recv.at[slot],
            device_id=(right,),
            device_id_type=pl.DeviceIdType.MESH,
        )
        rb = pltpu.make_async_remote_copy(
            src_ref=out_ref.at[pl.ds(cl * CHUNK + HALF, HALF), :],
            dst_ref=out_ref.at[pl.ds(cl * CHUNK + HALF, HALF), :],
            send_sem=sl_send.at[slot],
            recv_sem=sl_recv.at[slot],
            device_id=(left,),
            device_id_type=pl.DeviceIdType.MESH,
        )
        ra.start()
        rb.start()
        ra.wait()
        rb.wait()


def kernel(x, Wq, K_ext, V_ext, Wo):
    my = lax.axis_index("i")
    x2d = x.reshape(SQ, D_MODEL)
    Wq_loc = lax.dynamic_slice(Wq, (0, my * (H_PER * DH)), (D_MODEL, H_PER * DH))
    Wo_loc = lax.dynamic_slice(Wo, (my * (H_PER * DH), 0), (H_PER * DH, D_MODEL))
    Wq_h = Wq_loc.reshape(D_MODEL, H_PER, DH).transpose(1, 0, 2) * SCALE
    Wo_h = Wo_loc.reshape(H_PER, DH, D_MODEL)
    K_t = K_ext.reshape(SKV, H_PER, DH).transpose(1, 0, 2)
    V_t = V_ext.reshape(SKV, H_PER, DH).transpose(1, 0, 2)

    xp = x2d[Q_PERM]
    pad = [(0, 0), (0, SKV_RES - SKV), (0, 0)]
    K_res = jnp.pad(K_t[:, K_RES_SEL, :], pad)
    V_res = jnp.pad(V_t[:, K_RES_SEL, :], pad)

    reduced = pl.pallas_call(
        _fused_body,
        out_shape=jax.ShapeDtypeStruct((SQP, D_MODEL), jnp.float32),
        in_specs=[pl.BlockSpec(memory_space=pltpu.VMEM)] * 5,
        out_specs=pl.BlockSpec(memory_space=pltpu.VMEM),
        scratch_shapes=[
            pltpu.VMEM((2, CHUNK, D_MODEL), jnp.float32),
            pltpu.SemaphoreType.DMA((2,)),
            pltpu.SemaphoreType.DMA((2,)),
            pltpu.SemaphoreType.REGULAR,
        ],
        compiler_params=pltpu.CompilerParams(
            vmem_limit_bytes=110 * 1024 * 1024,
        ),
    )(xp, Wq_h, K_res, V_res, Wo_h)

    outp = pl.pallas_call(
        _ag_body,
        out_shape=jax.ShapeDtypeStruct((SQP, D_MODEL), jnp.float32),
        in_specs=[pl.BlockSpec(memory_space=pltpu.VMEM)],
        out_specs=pl.BlockSpec(memory_space=pltpu.VMEM),
        scratch_shapes=[
            pltpu.SemaphoreType.DMA((2,)),
            pltpu.SemaphoreType.DMA((2,)),
            pltpu.SemaphoreType.DMA((2,)),
            pltpu.SemaphoreType.DMA((2,)),
        ],
    )(reduced)
    return outp[INV_PERM].reshape(1, SQ, D_MODEL)


# device time: 233904 ns/iter; 3.9265x vs baseline; 3.9265x over previous
import numpy as _np

import jax
import jax.numpy as jnp
from jax import lax
from jax.experimental import pallas as pl
from jax.experimental.pallas import tpu as pltpu

N_DEV = 4
SQ = 2048
SKV = 2048
D_MODEL = 1024
H_PER = 8
DH = 128
BLK = 64
NBLK = SKV // BLK
SCALE = 0.08838834764831843

_G = [[b for b in range(NBLK) if b % 3 == r] for r in range(3)]
_P = [_G[0] + [0], _G[1] + [0], _G[2] + [0, 0]]
_PERM_BLOCKS = _P[0] + _P[1] + _P[2]
Q_PERM = _np.concatenate(
    [_np.arange(b * BLK, (b + 1) * BLK) for b in _PERM_BLOCKS]
)
_first: dict[int, int] = {}
for _pos, _rw in enumerate(Q_PERM):
    _first.setdefault(int(_rw), _pos)
INV_PERM = _np.array([_first[i] for i in range(SQ)])

SQP = len(Q_PERM)
CHUNK = SQP // N_DEV
HALF = CHUNK // 2

_RES_BLOCKS = _G[0] + _G[1] + _G[2]
SKV_RES = 2048 + 2 * BLK
M_LO = [0, 1408, 704]
M_W = [704, 640, 704]
D_OFF = [0, 704, 1408]
GRP_START = [0, 768, 1536]

SEGS = {
    0: [(0, 576, 0)],
    1: [(576, 192, 0), (768, 384, 1)],
    2: [(1152, 384, 1), (1536, 192, 2)],
    3: [(1728, 576, 2)],
}


def _fused_body(x_ref, wq_ref, k_ref, v_ref, wo_ref,
                out_ref, comm_ref, send_sems, recv_sems, credit_sem):
    my = lax.axis_index("i")
    right = (my + 1) % N_DEV
    left = (my - 1) % N_DEV

    def seg_compute(i0, R, r):
        m_lo, m_w = M_LO[r], M_W[r]
        d_lo = D_OFF[r] + (i0 - GRP_START[r])
        nb = R // BLK
        xb = x_ref[i0:i0 + R, :]
        out_ref[i0:i0 + R, :] = jnp.zeros((R, D_MODEL), jnp.float32)

        def h_body(h, carry):
            q = jnp.dot(xb, wq_ref[h], preferred_element_type=jnp.float32)
            kg = k_ref[h, m_lo:m_lo + m_w, :]
            vg = v_ref[h, m_lo:m_lo + m_w, :]
            sA = lax.dot_general(
                q, kg, (((1,), (1,)), ((), ())),
                preferred_element_type=jnp.float32,
            )
            eA = jnp.exp(sA)
            denom = jnp.sum(eA, axis=1, keepdims=True)
            ctx = jnp.dot(eA, vg, preferred_element_type=jnp.float32)
            if r != 0:
                s0 = lax.dot_general(
                    q, k_ref[h, 0:BLK, :], (((1,), (1,)), ((), ())),
                    preferred_element_type=jnp.float32,
                )
                e0 = jnp.exp(s0)
                denom = denom + jnp.sum(e0, axis=1, keepdims=True)
                ctx = ctx + jnp.dot(e0, v_ref[h, 0:BLK, :],
                                    preferred_element_type=jnp.float32)
                qd = q.reshape(nb, BLK, DH)
                kd = k_ref[h, d_lo:d_lo + R, :].reshape(nb, BLK, DH)
                vd = v_ref[h, d_lo:d_lo + R, :].reshape(nb, BLK, DH)
                sB = lax.dot_general(
                    qd, kd, (((2,), (2,)), ((0,), (0,))),
                    preferred_element_type=jnp.float32,
                )
                eB = jnp.exp(sB)
                denom = denom + jnp.sum(eB, axis=2).reshape(R, 1)
                ctx = ctx + lax.dot_general(
                    eB, vd, (((2,), (1,)), ((0,), (0,))),
                    preferred_element_type=jnp.float32,
                ).reshape(R, DH)
            ctx = ctx / denom
            out_ref[i0:i0 + R, :] += jnp.dot(
                ctx, wo_ref[h], preferred_element_type=jnp.float32
            )
            return carry

        lax.fori_loop(0, H_PER, h_body, None)

    def compute_chunk(c):
        for kk in range(N_DEV):
            @pl.when(c == kk)
            def _(kk=kk):
                for (i0, R, r) in SEGS[kk]:
                    seg_compute(i0, R, r)

    sends = []

    c0 = (my + 3) % N_DEV
    compute_chunk(c0)
    send0 = pltpu.make_async_remote_copy(
        src_ref=out_ref.at[pl.ds(c0 * CHUNK, CHUNK), :],
        dst_ref=comm_ref.at[0],
        send_sem=send_sems.at[0],
        recv_sem=recv_sems.at[0],
        device_id=(right,),
        device_id_type=pl.DeviceIdType.MESH,
    )
    send0.start()
    sends.append(send0)

    for s in range(1, N_DEV):
        c = (my + 3 - s) % N_DEV
        compute_chunk(c)
        rows = pl.ds(c * CHUNK, CHUNK)
        slot = (s - 1) % 2
        recv = pltpu.make_async_remote_copy(
            src_ref=comm_ref.at[slot],
            dst_ref=comm_ref.at[slot],
            send_sem=send_sems.at[slot],
            recv_sem=recv_sems.at[slot],
            device_id=(left,),
            device_id_type=pl.DeviceIdType.MESH,
        )
        recv.wait_recv()
        out_ref[rows, :] += comm_ref[slot]
        if s == 1:
            pl.semaphore_signal(
                credit_sem, inc=1,
                device_id=(left,), device_id_type=pl.DeviceIdType.MESH,
            )
        if s < N_DEV - 1:
            if s == 2:
                pl.semaphore_wait(credit_sem, 1)
                send0.wait_send()
            snd = pltpu.make_async_remote_copy(
                src_ref=out_ref.at[rows, :],
                dst_ref=comm_ref.at[s % 2],
                send_sem=send_sems.at[s % 2],
                recv_sem=recv_sems.at[s % 2],
                device_id=(right,),
                device_id_type=pl.DeviceIdType.MESH,
            )
            snd.start()
            sends.append(snd)

    sends[1].wait_send()
    sends[2].wait_send()


def _ag_body(p_ref, out_ref, sr_send, sr_recv, sl_send, sl_recv):
    my = lax.axis_index("i")
    right = (my + 1) % N_DEV
    left = (my + 3) % N_DEV

    out_ref[...] = p_ref[...]
    for t in range(N_DEV - 1):
        cr = (my - t) % N_DEV
        cl = (my + t) % N_DEV
        slot = t % 2
        ra = pltpu.make_async_remote_copy(
            src_ref=out_ref.at[pl.ds(cr * CHUNK, HALF), :],
            dst_ref=out_ref.at[pl.ds(cr * CHUNK, HALF), :],
            send_sem=sr_send.at[slot],
            recv_sem=sr_recv.at[slot],
            device_id=(right,),
            device_id_type=pl.DeviceIdType.MESH,
        )
        rb = pltpu.make_async_remote_copy(
            src_ref=out_ref.at[pl.ds(cl * CHUNK + HALF, HALF), :],
            dst_ref=out_ref.at[pl.ds(cl * CHUNK + HALF, HALF), :],
            send_sem=sl_send.at[slot],
            recv_sem=sl_recv.at[slot],
            device_id=(left,),
            device_id_type=pl.DeviceIdType.MESH,
        )
        ra.start()
        rb.start()
        ra.wait()
        rb.wait()


def kernel(x, Wq, K_ext, V_ext, Wo):
    my = lax.axis_index("i")
    x2d = x.reshape(SQ, D_MODEL)
    Wq_loc = lax.dynamic_slice(Wq, (0, my * (H_PER * DH)), (D_MODEL, H_PER * DH))
    Wo_loc = lax.dynamic_slice(Wo, (my * (H_PER * DH), 0), (H_PER * DH, D_MODEL))
    Wq_h = Wq_loc.reshape(D_MODEL, H_PER, DH).transpose(1, 0, 2) * SCALE
    Wo_h = Wo_loc.reshape(H_PER, DH, D_MODEL)
    K_t = K_ext.reshape(SKV, H_PER, DH).transpose(1, 0, 2)
    V_t = V_ext.reshape(SKV, H_PER, DH).transpose(1, 0, 2)

    xp = jnp.concatenate(
        [x2d[b * BLK:(b + 1) * BLK] for b in _PERM_BLOCKS], axis=0
    )
    _ztail = jnp.zeros((H_PER, SKV_RES - SKV, DH), jnp.float32)
    K_res = jnp.concatenate(
        [K_t[:, b * BLK:(b + 1) * BLK, :] for b in _RES_BLOCKS] + [_ztail],
        axis=1,
    )
    V_res = jnp.concatenate(
        [V_t[:, b * BLK:(b + 1) * BLK, :] for b in _RES_BLOCKS] + [_ztail],
        axis=1,
    )

    reduced = pl.pallas_call(
        _fused_body,
        out_shape=jax.ShapeDtypeStruct((SQP, D_MODEL), jnp.float32),
        in_specs=[pl.BlockSpec(memory_space=pltpu.VMEM)] * 5,
        out_specs=pl.BlockSpec(memory_space=pltpu.VMEM),
        scratch_shapes=[
            pltpu.VMEM((2, CHUNK, D_MODEL), jnp.float32),
            pltpu.SemaphoreType.DMA((2,)),
            pltpu.SemaphoreType.DMA((2,)),
            pltpu.SemaphoreType.REGULAR,
        ],
        compiler_params=pltpu.CompilerParams(
            vmem_limit_bytes=110 * 1024 * 1024,
        ),
    )(xp, Wq_h, K_res, V_res, Wo_h)

    outp = pl.pallas_call(
        _ag_body,
        out_shape=jax.ShapeDtypeStruct((SQP, D_MODEL), jnp.float32),
        in_specs=[pl.BlockSpec(memory_space=pltpu.VMEM)],
        out_specs=pl.BlockSpec(memory_space=pltpu.VMEM),
        scratch_shapes=[
            pltpu.SemaphoreType.DMA((2,)),
            pltpu.SemaphoreType.DMA((2,)),
            pltpu.SemaphoreType.DMA((2,)),
            pltpu.SemaphoreType.DMA((2,)),
        ],
    )(reduced)
    _inv_blocks = [int(INV_PERM[b * BLK]) // BLK for b in range(NBLK)]
    out2d = jnp.concatenate(
        [outp[p * BLK:(p + 1) * BLK] for p in _inv_blocks], axis=0
    )
    return out2d.reshape(1, SQ, D_MODEL)


# device time: 185942 ns/iter; 4.9393x vs baseline; 1.2579x over previous
import jax
import jax.numpy as jnp
from jax import lax
from jax.experimental import pallas as pl
from jax.experimental.pallas import tpu as pltpu

N_DEV = 4
SQ = 2048
SKV = 2048
D_MODEL = 1024
H_PER = 8
DH = 128
CHUNK = SQ // N_DEV
HALF = CHUNK // 2
SCALE = 0.08838834764831843


def _fused_body(x_ref, wq_ref, k_ref, v_ref, wo_ref, out_ref,
                bias_ref, comm_ref, send_sems, recv_sems, credit_sem,
                sr_send, sr_recv, sl_send, sl_recv):
    my = lax.axis_index("i")
    right = (my + 1) % N_DEV
    left = (my - 1) % N_DEV

    def compute_chunk(c):
        rows = pl.ds(c * CHUNK, CHUNK)
        row = c * CHUNK + lax.broadcasted_iota(jnp.int32, (CHUNK, SKV), 0)
        col = lax.broadcasted_iota(jnp.int32, (CHUNK, SKV), 1)
        qb = row // 64
        kb = col // 64
        mask = (qb == kb) | (kb == 0) | (((qb + kb) % 3) == 0)
        bias_ref[...] = jnp.where(mask, 0.0, -1e9)

        xb = x_ref[rows, :]
        out_ref[rows, :] = jnp.zeros((CHUNK, D_MODEL), jnp.float32)

        def h_body(h, _):
            q = jnp.dot(xb, wq_ref[h], preferred_element_type=jnp.float32)
            s = lax.dot_general(
                q, k_ref[h], (((1,), (1,)), ((), ())),
                preferred_element_type=jnp.float32,
            ) + bias_ref[...]
            w = jnp.exp(s)
            denom = jnp.sum(w, axis=1, keepdims=True)
            ctx = jnp.dot(w, v_ref[h], preferred_element_type=jnp.float32)
            ctx = ctx / denom
            out_ref[rows, :] += jnp.dot(ctx, wo_ref[h],
                                        preferred_element_type=jnp.float32)
            return _

        lax.fori_loop(0, H_PER, h_body, None)

    sends = []

    c0 = (my + 3) % N_DEV
    compute_chunk(c0)
    send0 = pltpu.make_async_remote_copy(
        src_ref=out_ref.at[pl.ds(c0 * CHUNK, CHUNK), :],
        dst_ref=comm_ref.at[0],
        send_sem=send_sems.at[0],
        recv_sem=recv_sems.at[0],
        device_id=(right,),
        device_id_type=pl.DeviceIdType.MESH,
    )
    send0.start()
    sends.append(send0)

    for s in range(1, N_DEV):
        c = (my + 3 - s) % N_DEV
        compute_chunk(c)
        rows = pl.ds(c * CHUNK, CHUNK)
        slot = (s - 1) % 2
        recv = pltpu.make_async_remote_copy(
            src_ref=comm_ref.at[slot],
            dst_ref=comm_ref.at[slot],
            send_sem=send_sems.at[slot],
            recv_sem=recv_sems.at[slot],
            device_id=(left,),
            device_id_type=pl.DeviceIdType.MESH,
        )
        recv.wait_recv()
        out_ref[rows, :] += comm_ref[slot]
        if s == 1:
            pl.semaphore_signal(
                credit_sem, inc=1,
                device_id=(left,), device_id_type=pl.DeviceIdType.MESH,
            )
        if s < N_DEV - 1:
            if s == 2:
                pl.semaphore_wait(credit_sem, 1)
                send0.wait_send()
            snd = pltpu.make_async_remote_copy(
                src_ref=out_ref.at[rows, :],
                dst_ref=comm_ref.at[s % 2],
                send_sem=send_sems.at[s % 2],
                recv_sem=recv_sems.at[s % 2],
                device_id=(right,),
                device_id_type=pl.DeviceIdType.MESH,
            )
            snd.start()
            sends.append(snd)

    for t in range(N_DEV - 1):
        cr = (my - t) % N_DEV
        cl = (my + t) % N_DEV
        slot = t % 2
        ra = pltpu.make_async_remote_copy(
            src_ref=out_ref.at[pl.ds(cr * CHUNK, HALF), :],
            dst_ref=out_ref.at[pl.ds(cr * CHUNK, HALF), :],
            send_sem=sr_send.at[slot],
            recv_sem=sr_recv.at[slot],
            device_id=(right,),
            device_id_type=pl.DeviceIdType.MESH,
        )
        rb = pltpu.make_async_remote_copy(
            src_ref=out_ref.at[pl.ds(cl * CHUNK + HALF, HALF), :],
            dst_ref=out_ref.at[pl.ds(cl * CHUNK + HALF, HALF), :],
            send_sem=sl_send.at[slot],
            recv_sem=sl_recv.at[slot],
            device_id=(left,),
            device_id_type=pl.DeviceIdType.MESH,
        )
        ra.start()
        rb.start()
        ra.wait()
        rb.wait()

    sends[1].wait_send()
    sends[2].wait_send()


def kernel(x, Wq, K_ext, V_ext, Wo):
    my = lax.axis_index("i")
    x2d = x.reshape(SQ, D_MODEL)
    Wq_loc = lax.dynamic_slice(Wq, (0, my * (H_PER * DH)), (D_MODEL, H_PER * DH))
    Wo_loc = lax.dynamic_slice(Wo, (my * (H_PER * DH), 0), (H_PER * DH, D_MODEL))
    Wq_h = Wq_loc.reshape(D_MODEL, H_PER, DH).transpose(1, 0, 2) * SCALE
    Wo_h = Wo_loc.reshape(H_PER, DH, D_MODEL)
    K = K_ext.reshape(SKV, H_PER, DH).transpose(1, 0, 2)
    V = V_ext.reshape(SKV, H_PER, DH).transpose(1, 0, 2)

    out = pl.pallas_call(
        _fused_body,
        out_shape=jax.ShapeDtypeStruct((SQ, D_MODEL), jnp.float32),
        in_specs=[pl.BlockSpec(memory_space=pltpu.VMEM)] * 5,
        out_specs=pl.BlockSpec(memory_space=pltpu.VMEM),
        scratch_shapes=[
            pltpu.VMEM((CHUNK, SKV), jnp.float32),
            pltpu.VMEM((2, CHUNK, D_MODEL), jnp.float32),
            pltpu.SemaphoreType.DMA((2,)),
            pltpu.SemaphoreType.DMA((2,)),
            pltpu.SemaphoreType.REGULAR,
            pltpu.SemaphoreType.DMA((2,)),
            pltpu.SemaphoreType.DMA((2,)),
            pltpu.SemaphoreType.DMA((2,)),
            pltpu.SemaphoreType.DMA((2,)),
        ],
        compiler_params=pltpu.CompilerParams(
            vmem_limit_bytes=100 * 1024 * 1024,
        ),
    )(x2d, Wq_h, K, V, Wo_h)
    return out.reshape(1, SQ, D_MODEL)
